# baseline (speedup 1.0000x reference)
import jax
import jax.numpy as jnp
from jax import lax
from jax.experimental import pallas as pl
from jax.experimental.pallas import tpu as pltpu
from jax.experimental.pallas import tpu_sc as plsc

N = 10000
E = 320000
D = 128
U = 64

NC = 2
NS = 16
L = 16
NW = NC * NS
EPW = E // NW
CH = 80
NCH = EPW // CH
WROW = U + L


def _prep_body(x_ref, w_ref, asrc_ref, adst_ref, mext_ref, asd_ref, c_ref):
    m = jnp.dot(x_ref[...], w_ref[...], preferred_element_type=jnp.float32)
    a_s = jnp.sum(m * asrc_ref[...], axis=1)
    a_d = jnp.sum(m * adst_ref[...], axis=1)
    asd_ref[0, :] = a_s
    asd_ref[1, :] = a_d
    cb = jnp.max(a_s) + jnp.max(a_d)
    c_ref[0, 0] = jnp.where(cb > 0.0, cb, 0.2 * cb)
    n = x_ref.shape[0]
    pad = jnp.concatenate(
        [jnp.ones((n, 1), jnp.float32), jnp.zeros((n, L - 1), jnp.float32)],
        axis=1)
    mext_ref[...] = jnp.concatenate([m, pad], axis=1)


def _prep(x, w, asrc, adst):
    return pl.pallas_call(
        _prep_body,
        out_shape=[
            jax.ShapeDtypeStruct((N, WROW), jnp.float32),
            jax.ShapeDtypeStruct((2, N), jnp.float32),
            jax.ShapeDtypeStruct((1, 1), jnp.float32),
        ],
        out_specs=[
            pl.BlockSpec(memory_space=pltpu.VMEM),
            pl.BlockSpec(memory_space=pltpu.VMEM),
            pl.BlockSpec(memory_space=pltpu.SMEM),
        ],
    )(x, w, asrc, adst)


def _edge_body(mext_hbm, asrc_hbm, adst_hbm, cvec_hbm, srcr_hbm, dstr_hbm,
               out_hbm,
               a_s_v, a_d_v, src_v, dst_v, ee_v, rows_v, rows2_v, rows3_v,
               c_v, agg_sh, gsem, gsem2, gsem3, ssem, ssem2, ssem3):
    cid = lax.axis_index("c")
    sid = lax.axis_index("s")
    wid = cid * NS + sid

    pltpu.sync_copy(asrc_hbm, a_s_v)
    pltpu.sync_copy(adst_hbm, a_d_v)
    pltpu.sync_copy(srcr_hbm.at[wid], src_v)
    pltpu.sync_copy(dstr_hbm.at[wid], dst_v)
    pltpu.sync_copy(cvec_hbm, c_v)
    cvec = c_v[...]

    @plsc.parallel_loop(0, NCH, unroll=4)
    def _(r):
        for p in range(WROW // L):
            ee_v[r, pl.ds(p * L, L)] = jnp.zeros((L,), jnp.float32)

    rows_per_tile = N // NS
    for t in range(rows_per_tile // NCH):
        pltpu.sync_copy(
            ee_v, agg_sh.at[pl.ds(sid * rows_per_tile + t * NCH, NCH)])
    plsc.subcore_barrier()

    @plsc.parallel_loop(0, NCH, unroll=2)
    def _(r):
        for p in range(CH // L):
            sl = pl.ds(p * L, L)
            si = src_v[r, sl]
            di = dst_v[r, sl]
            sv = plsc.load_gather(a_s_v, [si])
            dv = plsc.load_gather(a_d_v, [di])
            e = sv + dv
            e = jnp.where(e >= 0.0, e, 0.2 * e)
            ee_v[r, sl] = jnp.exp(e - cvec)

    bufs = (rows_v, rows2_v, rows3_v)
    gsems = (gsem, gsem2, gsem3)
    ssems = (ssem, ssem2, ssem3)

    def g_issue(j, b):
        pltpu.async_copy(mext_hbm.at[src_v.at[j]], bufs[b], gsems[b])

    def s_wait(j, b):
        pltpu.make_async_copy(bufs[b], agg_sh.at[dst_v.at[j]],
                              ssems[b]).wait()

    def step(j, b, first=False, last_d=False):
        pltpu.make_async_copy(mext_hbm.at[src_v.at[j]], bufs[b],
                              gsems[b]).wait()
        rbuf = bufs[b]

        @plsc.parallel_loop(0, CH, unroll=8)
        def _(k):
            eek = plsc.load_gather(ee_v, [jnp.full((L,), j, jnp.int32),
                                          jnp.full((L,), k, jnp.int32)])
            for p in range(WROW // L):
                sl = pl.ds(p * L, L)
                rbuf[k, sl] = rbuf[k, sl] * eek

        pltpu.async_copy(rbuf, agg_sh.at[dst_v.at[j]], ssems[b], add=True)
        if not first:
            s_wait(j - 1, (b + 2) % 3)
        if last_d:
            @pl.when(j + 2 < NCH)
            def _():
                g_issue(j + 2, (b + 2) % 3)
        else:
            g_issue(j + 2, (b + 2) % 3)

    g_issue(0, 0)
    g_issue(1, 1)
    step(0, 0, first=True)

    @pl.loop(0, (NCH - 2) // 3)
    def _(t):
        j = 1 + 3 * t
        step(j, 1)
        step(j + 1, 2)
        step(j + 2, 0, last_d=True)

    step(NCH - 1, (NCH - 1) % 3, last_d=True)
    s_wait(NCH - 1, (NCH - 1) % 3)

    plsc.subcore_barrier()

    for t in range(rows_per_tile // NCH):
        base = sid * rows_per_tile + t * NCH
        pltpu.sync_copy(agg_sh.at[pl.ds(base, NCH)], ee_v)
        pltpu.sync_copy(ee_v, out_hbm.at[cid].at[pl.ds(base, NCH)])


_edge = pl.kernel(
    _edge_body,
    out_type=jax.ShapeDtypeStruct((NC, N, WROW), jnp.float32),
    mesh=plsc.VectorSubcoreMesh(core_axis_name="c", subcore_axis_name="s",
                                num_cores=NC, num_subcores=NS),
    scratch_types=[
        pltpu.VMEM((N,), jnp.float32),
        pltpu.VMEM((N,), jnp.float32),
        pltpu.VMEM((NCH, CH), jnp.int32),
        pltpu.VMEM((NCH, CH), jnp.int32),
        pltpu.VMEM((NCH, CH), jnp.float32),
        pltpu.VMEM((CH, WROW), jnp.float32),
        pltpu.VMEM((CH, WROW), jnp.float32),
        pltpu.VMEM((CH, WROW), jnp.float32),
        pltpu.VMEM((L,), jnp.float32),
        pltpu.VMEM_SHARED((N, WROW), jnp.float32),
        pltpu.SemaphoreType.DMA,
        pltpu.SemaphoreType.DMA,
        pltpu.SemaphoreType.DMA,
        pltpu.SemaphoreType.DMA,
        pltpu.SemaphoreType.DMA,
        pltpu.SemaphoreType.DMA,
    ],
    compiler_params=pltpu.CompilerParams(use_tc_tiling_on_sc=False,
                                         needs_layout_passes=False),
)


def _gru_body(parts_ref, h_ref, wr_ref, wu_ref, wc_ref, br_ref, bu_ref,
              bc_ref, out_ref):
    s = parts_ref[0] + parts_ref[1]
    denom = s[:, U:U + 1] + 1e-16
    agg = s[:, 0:U] / denom
    h = h_ref[...]
    xr = jnp.concatenate([agg, h], axis=1)
    r = jax.nn.sigmoid(
        jnp.dot(xr, wr_ref[...], preferred_element_type=jnp.float32)
        + br_ref[...])
    u = jax.nn.sigmoid(
        jnp.dot(xr, wu_ref[...], preferred_element_type=jnp.float32)
        + bu_ref[...])
    xc = jnp.concatenate([agg, r * h], axis=1)
    c = jnp.tanh(
        jnp.dot(xc, wc_ref[...], preferred_element_type=jnp.float32)
        + bc_ref[...])
    out_ref[...] = u * h + (1.0 - u) * c


def _gru(parts, h, wr, wu, wc, br, bu, bc):
    return pl.pallas_call(
        _gru_body,
        out_shape=jax.ShapeDtypeStruct((N, U), jnp.float32),
    )(parts, h, wr, wu, wc, br, bu, bc)


def kernel(x, h, edge_index, W, att_src, att_dst, Wr, Wu, Wc, br, bu, bc):
    mext, asd, cout = _prep(x, W, att_src.reshape(1, U), att_dst.reshape(1, U))
    cvec = jnp.full((L,), cout[0, 0], jnp.float32)
    srcr = edge_index[0].reshape(NW, NCH, CH)
    dstr = edge_index[1].reshape(NW, NCH, CH)
    parts = _edge(mext, asd[0], asd[1], cvec, srcr, dstr)
    return _gru(parts, h, Wr, Wu, Wc, br.reshape(1, U), bu.reshape(1, U),
                bc.reshape(1, U))


# speedup vs baseline: 1.0374x; 1.0374x over previous
import jax
import jax.numpy as jnp
from jax import lax
from jax.experimental import pallas as pl
from jax.experimental.pallas import tpu as pltpu
from jax.experimental.pallas import tpu_sc as plsc

N = 10000
E = 320000
D = 128
U = 64

NC = 2
NS = 16
L = 16
NW = NC * NS
EPW = E // NW
CH = 80
NCH = EPW // CH
WROW = U


def _prep_body(x_ref, w_ref, asrc_ref, adst_ref, mext_ref, asd_ref, c_ref):
    m = jnp.dot(x_ref[...], w_ref[...], preferred_element_type=jnp.float32)
    a_s = jnp.sum(m * asrc_ref[...], axis=1)
    a_d = jnp.sum(m * adst_ref[...], axis=1)
    asd_ref[0, :] = a_s
    asd_ref[1, :] = a_d
    cb = jnp.max(a_s) + jnp.max(a_d)
    c_ref[0, 0] = jnp.where(cb > 0.0, cb, 0.2 * cb)
    mext_ref[...] = m


def _prep(x, w, asrc, adst):
    return pl.pallas_call(
        _prep_body,
        out_shape=[
            jax.ShapeDtypeStruct((N, WROW), jnp.float32),
            jax.ShapeDtypeStruct((2, N), jnp.float32),
            jax.ShapeDtypeStruct((1, 1), jnp.float32),
        ],
        out_specs=[
            pl.BlockSpec(memory_space=pltpu.VMEM),
            pl.BlockSpec(memory_space=pltpu.VMEM),
            pl.BlockSpec(memory_space=pltpu.SMEM),
        ],
    )(x, w, asrc, adst)


def _edge_body(mext_hbm, asrc_hbm, adst_hbm, cvec_hbm, srcr_hbm, dstr_hbm,
               out_hbm, outd_hbm,
               a_s_v, a_d_v, src_v, dst_v, ee_v, rows_v, rows2_v, rows3_v,
               denom_v, obuf_v, c_v, agg_sh,
               gsem, gsem2, gsem3, ssem, ssem2, ssem3):
    cid = lax.axis_index("c")
    sid = lax.axis_index("s")
    wid = cid * NS + sid

    pltpu.sync_copy(asrc_hbm, a_s_v)
    pltpu.sync_copy(adst_hbm, a_d_v)
    pltpu.sync_copy(srcr_hbm.at[wid], src_v)
    pltpu.sync_copy(dstr_hbm.at[wid], dst_v)
    pltpu.sync_copy(cvec_hbm, c_v)
    cvec = c_v[...]

    @plsc.parallel_loop(0, NCH, unroll=4)
    def _(r):
        for p in range(WROW // L):
            obuf_v[r, pl.ds(p * L, L)] = jnp.zeros((L,), jnp.float32)

    @plsc.parallel_loop(0, N // L, unroll=4)
    def _(r):
        denom_v[pl.ds(r * L, L)] = jnp.zeros((L,), jnp.float32)

    rows_per_tile = N // NS
    for t in range(rows_per_tile // NCH):
        pltpu.sync_copy(
            obuf_v, agg_sh.at[pl.ds(sid * rows_per_tile + t * NCH, NCH)])
    plsc.subcore_barrier()

    @plsc.parallel_loop(0, NCH, unroll=2)
    def _(r):
        for p in range(CH // L):
            sl = pl.ds(p * L, L)
            si = src_v[r, sl]
            di = dst_v[r, sl]
            sv = plsc.load_gather(a_s_v, [si])
            dv = plsc.load_gather(a_d_v, [di])
            e = sv + dv
            e = jnp.where(e >= 0.0, e, 0.2 * e)
            ee = jnp.exp(e - cvec)
            ee_v[r, sl] = ee
            plsc.addupdate_scatter(denom_v, [di], ee)

    bufs = (rows_v, rows2_v, rows3_v)
    gsems = (gsem, gsem2, gsem3)
    ssems = (ssem, ssem2, ssem3)

    def g_issue(j, b):
        pltpu.async_copy(mext_hbm.at[src_v.at[j]], bufs[b], gsems[b])

    def s_wait(j, b):
        pltpu.make_async_copy(bufs[b], agg_sh.at[dst_v.at[j]],
                              ssems[b]).wait()

    def step(j, b, first=False, last_d=False):
        pltpu.make_async_copy(mext_hbm.at[src_v.at[j]], bufs[b],
                              gsems[b]).wait()
        rbuf = bufs[b]

        @plsc.parallel_loop(0, CH, unroll=4)
        def _(k):
            eek = plsc.load_gather(ee_v, [jnp.full((L,), j, jnp.int32),
                                          jnp.full((L,), k, jnp.int32)])
            for p in range(WROW // L):
                sl = pl.ds(p * L, L)
                rbuf[k, sl] = rbuf[k, sl] * eek

        pltpu.async_copy(rbuf, agg_sh.at[dst_v.at[j]], ssems[b], add=True)
        if not first:
            s_wait(j - 1, (b + 2) % 3)
        if last_d:
            @pl.when(j + 2 < NCH)
            def _():
                g_issue(j + 2, (b + 2) % 3)
        else:
            g_issue(j + 2, (b + 2) % 3)

    g_issue(0, 0)
    g_issue(1, 1)
    step(0, 0, first=True)

    @pl.loop(0, (NCH - 2) // 3)
    def _(t):
        j = 1 + 3 * t
        step(j, 1)
        step(j + 1, 2)
        step(j + 2, 0, last_d=True)

    step(NCH - 1, (NCH - 1) % 3, last_d=True)
    s_wait(NCH - 1, (NCH - 1) % 3)

    plsc.subcore_barrier()

    pltpu.sync_copy(denom_v, outd_hbm.at[wid])
    for t in range(rows_per_tile // NCH):
        base = sid * rows_per_tile + t * NCH
        pltpu.sync_copy(agg_sh.at[pl.ds(base, NCH)], obuf_v)
        pltpu.sync_copy(obuf_v, out_hbm.at[cid].at[pl.ds(base, NCH)])


_edge = pl.kernel(
    _edge_body,
    out_type=[jax.ShapeDtypeStruct((NC, N, WROW), jnp.float32),
              jax.ShapeDtypeStruct((NW, N), jnp.float32)],
    mesh=plsc.VectorSubcoreMesh(core_axis_name="c", subcore_axis_name="s",
                                num_cores=NC, num_subcores=NS),
    scratch_types=[
        pltpu.VMEM((N,), jnp.float32),
        pltpu.VMEM((N,), jnp.float32),
        pltpu.VMEM((NCH, CH), jnp.int32),
        pltpu.VMEM((NCH, CH), jnp.int32),
        pltpu.VMEM((NCH, CH), jnp.float32),
        pltpu.VMEM((CH, WROW), jnp.float32),
        pltpu.VMEM((CH, WROW), jnp.float32),
        pltpu.VMEM((CH, WROW), jnp.float32),
        pltpu.VMEM((N,), jnp.float32),
        pltpu.VMEM((NCH, WROW), jnp.float32),
        pltpu.VMEM((L,), jnp.float32),
        pltpu.VMEM_SHARED((N, WROW), jnp.float32),
        pltpu.SemaphoreType.DMA,
        pltpu.SemaphoreType.DMA,
        pltpu.SemaphoreType.DMA,
        pltpu.SemaphoreType.DMA,
        pltpu.SemaphoreType.DMA,
        pltpu.SemaphoreType.DMA,
    ],
    compiler_params=pltpu.CompilerParams(use_tc_tiling_on_sc=False,
                                         needs_layout_passes=False),
)


def _gru_body(parts_ref, dpt_ref, h_ref, wr_ref, wu_ref, wc_ref, br_ref,
              bu_ref, bc_ref, out_ref):
    s = parts_ref[0] + parts_ref[1]
    denom = jnp.sum(dpt_ref[...], axis=1, keepdims=True) + 1e-16
    agg = s / denom
    h = h_ref[...]
    xr = jnp.concatenate([agg, h], axis=1)
    r = jax.nn.sigmoid(
        jnp.dot(xr, wr_ref[...], preferred_element_type=jnp.float32)
        + br_ref[...])
    u = jax.nn.sigmoid(
        jnp.dot(xr, wu_ref[...], preferred_element_type=jnp.float32)
        + bu_ref[...])
    xc = jnp.concatenate([agg, r * h], axis=1)
    c = jnp.tanh(
        jnp.dot(xc, wc_ref[...], preferred_element_type=jnp.float32)
        + bc_ref[...])
    out_ref[...] = u * h + (1.0 - u) * c


def _gru(parts, dpt, h, wr, wu, wc, br, bu, bc):
    return pl.pallas_call(
        _gru_body,
        out_shape=jax.ShapeDtypeStruct((N, U), jnp.float32),
    )(parts, dpt, h, wr, wu, wc, br, bu, bc)


def kernel(x, h, edge_index, W, att_src, att_dst, Wr, Wu, Wc, br, bu, bc):
    mext, asd, cout = _prep(x, W, att_src.reshape(1, U), att_dst.reshape(1, U))
    cvec = jnp.full((L,), cout[0, 0], jnp.float32)
    srcr = edge_index[0].reshape(NW, NCH, CH)
    dstr = edge_index[1].reshape(NW, NCH, CH)
    parts, dparts = _edge(mext, asd[0], asd[1], cvec, srcr, dstr)
    return _gru(parts, dparts.T, h, Wr, Wu, Wc, br.reshape(1, U),
                bu.reshape(1, U), bc.reshape(1, U))


# speedup vs baseline: 1.1541x; 1.1125x over previous
import jax
import jax.numpy as jnp
from jax import lax
from jax.experimental import pallas as pl
from jax.experimental.pallas import tpu as pltpu
from jax.experimental.pallas import tpu_sc as plsc

N = 10000
E = 320000
D = 128
U = 64

NC = 2
NS = 16
L = 16
NW = NC * NS
EPW = E // NW
CH = 80
NCH = EPW // CH
WROW = U


def _prep_body(x_ref, w_ref, asrc_ref, adst_ref, mext_ref, asd_ref):
    m = jnp.dot(x_ref[...], w_ref[...], preferred_element_type=jnp.float32)
    a_s = jnp.sum(m * asrc_ref[...], axis=1)
    a_d = jnp.sum(m * adst_ref[...], axis=1)
    asd_ref[0, :] = a_s
    asd_ref[1, :] = a_d
    cb = jnp.max(a_s) + jnp.max(a_d)
    asd_ref[2, :] = jnp.full((N,), jnp.where(cb > 0.0, cb, 0.2 * cb),
                             jnp.float32)
    mext_ref[...] = m


def _prep(x, w, asrc, adst):
    return pl.pallas_call(
        _prep_body,
        out_shape=[
            jax.ShapeDtypeStruct((N, WROW), jnp.float32),
            jax.ShapeDtypeStruct((3, N), jnp.float32),
        ],
        out_specs=[
            pl.BlockSpec(memory_space=pltpu.VMEM),
            pl.BlockSpec(memory_space=pltpu.VMEM),
        ],
    )(x, w, asrc, adst)


def _edge_body(mext_hbm, asd_hbm, sd_hbm,
               out_hbm, outd_hbm,
               a_s_v, a_d_v, src_v, dst_v, ee_v, rows_v, rows2_v, rows3_v,
               denom_v, obuf_v, c_v, agg_sh,
               gsem, gsem2, gsem3, ssem, ssem2, ssem3):
    cid = lax.axis_index("c")
    sid = lax.axis_index("s")
    wid = cid * NS + sid

    pltpu.async_copy(asd_hbm.at[0], a_s_v, gsem)
    pltpu.async_copy(asd_hbm.at[1], a_d_v, gsem2)
    pltpu.async_copy(sd_hbm.at[0].at[wid], src_v, gsem3)
    pltpu.async_copy(sd_hbm.at[1].at[wid], dst_v, ssem)
    pltpu.async_copy(asd_hbm.at[2].at[pl.ds(0, L)], c_v, ssem2)
    pltpu.make_async_copy(asd_hbm.at[0], a_s_v, gsem).wait()
    pltpu.make_async_copy(asd_hbm.at[1], a_d_v, gsem2).wait()
    pltpu.make_async_copy(sd_hbm.at[0].at[wid], src_v, gsem3).wait()
    pltpu.make_async_copy(sd_hbm.at[1].at[wid], dst_v, ssem).wait()
    pltpu.make_async_copy(asd_hbm.at[2].at[pl.ds(0, L)], c_v, ssem2).wait()
    cvec = c_v[...]

    @plsc.parallel_loop(0, NCH, unroll=4)
    def _(r):
        for p in range(WROW // L):
            obuf_v[r, pl.ds(p * L, L)] = jnp.zeros((L,), jnp.float32)

    @plsc.parallel_loop(0, N // L, unroll=4)
    def _(r):
        denom_v[pl.ds(r * L, L)] = jnp.zeros((L,), jnp.float32)

    rows_per_tile = N // NS
    for t in range(rows_per_tile // NCH):
        pltpu.sync_copy(
            obuf_v, agg_sh.at[pl.ds(sid * rows_per_tile + t * NCH, NCH)])
    plsc.subcore_barrier()

    @plsc.parallel_loop(0, NCH, unroll=2)
    def _(r):
        for p in range(CH // L):
            sl = pl.ds(p * L, L)
            si = src_v[r, sl]
            di = dst_v[r, sl]
            sv = plsc.load_gather(a_s_v, [si])
            dv = plsc.load_gather(a_d_v, [di])
            e = sv + dv
            e = jnp.where(e >= 0.0, e, 0.2 * e)
            ee = jnp.exp(e - cvec)
            ee_v[r, sl] = ee
            plsc.addupdate_scatter(denom_v, [di], ee)

    bufs = (rows_v, rows2_v, rows3_v)
    gsems = (gsem, gsem2, gsem3)
    ssems = (ssem, ssem2, ssem3)

    def g_issue(j, b):
        pltpu.async_copy(mext_hbm.at[src_v.at[j]], bufs[b], gsems[b])

    def s_wait(j, b):
        pltpu.make_async_copy(bufs[b], agg_sh.at[dst_v.at[j]],
                              ssems[b]).wait()

    def step(j, b, first=False, last_d=False):
        pltpu.make_async_copy(mext_hbm.at[src_v.at[j]], bufs[b],
                              gsems[b]).wait()
        rbuf = bufs[b]

        @plsc.parallel_loop(0, CH, unroll=4)
        def _(k):
            eek = plsc.load_gather(ee_v, [jnp.full((L,), j, jnp.int32),
                                          jnp.full((L,), k, jnp.int32)])
            for p in range(WROW // L):
                sl = pl.ds(p * L, L)
                rbuf[k, sl] = rbuf[k, sl] * eek

        pltpu.async_copy(rbuf, agg_sh.at[dst_v.at[j]], ssems[b], add=True)
        if not first:
            s_wait(j - 1, (b + 2) % 3)
        if last_d:
            @pl.when(j + 2 < NCH)
            def _():
                g_issue(j + 2, (b + 2) % 3)
        else:
            g_issue(j + 2, (b + 2) % 3)

    g_issue(0, 0)
    g_issue(1, 1)
    step(0, 0, first=True)

    @pl.loop(0, (NCH - 2) // 3)
    def _(t):
        j = 1 + 3 * t
        step(j, 1)
        step(j + 1, 2)
        step(j + 2, 0, last_d=True)

    step(NCH - 1, (NCH - 1) % 3, last_d=True)
    s_wait(NCH - 1, (NCH - 1) % 3)

    plsc.subcore_barrier()

    pltpu.async_copy(denom_v, outd_hbm.at[wid], gsem)
    for t in range(rows_per_tile // NCH):
        base = sid * rows_per_tile + t * NCH
        pltpu.async_copy(agg_sh.at[pl.ds(base, NCH)],
                         out_hbm.at[cid].at[pl.ds(base, NCH)], ssems[t % 3])
    pltpu.make_async_copy(denom_v, outd_hbm.at[wid], gsem).wait()
    for t in range(rows_per_tile // NCH):
        base = sid * rows_per_tile + t * NCH
        pltpu.make_async_copy(agg_sh.at[pl.ds(base, NCH)],
                              out_hbm.at[cid].at[pl.ds(base, NCH)],
                              ssems[t % 3]).wait()


_edge = pl.kernel(
    _edge_body,
    out_type=[jax.ShapeDtypeStruct((NC, N, WROW), jnp.float32),
              jax.ShapeDtypeStruct((NW, N), jnp.float32)],
    mesh=plsc.VectorSubcoreMesh(core_axis_name="c", subcore_axis_name="s",
                                num_cores=NC, num_subcores=NS),
    scratch_types=[
        pltpu.VMEM((N,), jnp.float32),
        pltpu.VMEM((N,), jnp.float32),
        pltpu.VMEM((NCH, CH), jnp.int32),
        pltpu.VMEM((NCH, CH), jnp.int32),
        pltpu.VMEM((NCH, CH), jnp.float32),
        pltpu.VMEM((CH, WROW), jnp.float32),
        pltpu.VMEM((CH, WROW), jnp.float32),
        pltpu.VMEM((CH, WROW), jnp.float32),
        pltpu.VMEM((N,), jnp.float32),
        pltpu.VMEM((NCH, WROW), jnp.float32),
        pltpu.VMEM((L,), jnp.float32),
        pltpu.VMEM_SHARED((N, WROW), jnp.float32),
        pltpu.SemaphoreType.DMA,
        pltpu.SemaphoreType.DMA,
        pltpu.SemaphoreType.DMA,
        pltpu.SemaphoreType.DMA,
        pltpu.SemaphoreType.DMA,
        pltpu.SemaphoreType.DMA,
    ],
    compiler_params=pltpu.CompilerParams(use_tc_tiling_on_sc=False,
                                         needs_layout_passes=False),
)


def _gru_body(parts_ref, dpt_ref, h_ref, wr_ref, wu_ref, wc_ref, br_ref,
              bu_ref, bc_ref, out_ref):
    s = parts_ref[0] + parts_ref[1]
    den = jnp.sum(dpt_ref[...], axis=0)
    denom = den.reshape(s.shape[0], 1) + 1e-16
    agg = s / denom
    h = h_ref[...]
    xr = jnp.concatenate([agg, h], axis=1)
    r = jax.nn.sigmoid(
        jnp.dot(xr, wr_ref[...], preferred_element_type=jnp.float32)
        + br_ref[...])
    u = jax.nn.sigmoid(
        jnp.dot(xr, wu_ref[...], preferred_element_type=jnp.float32)
        + bu_ref[...])
    xc = jnp.concatenate([agg, r * h], axis=1)
    c = jnp.tanh(
        jnp.dot(xc, wc_ref[...], preferred_element_type=jnp.float32)
        + bc_ref[...])
    out_ref[...] = u * h + (1.0 - u) * c


def _gru(parts, dpt, h, wr, wu, wc, br, bu, bc):
    return pl.pallas_call(
        _gru_body,
        out_shape=jax.ShapeDtypeStruct((N, U), jnp.float32),
    )(parts, dpt, h, wr, wu, wc, br, bu, bc)


def kernel(x, h, edge_index, W, att_src, att_dst, Wr, Wu, Wc, br, bu, bc):
    mext, asd = _prep(x, W, att_src.reshape(1, U), att_dst.reshape(1, U))
    sd = edge_index.reshape(2, NW, NCH, CH)
    parts, dparts = _edge(mext, asd, sd)
    return _gru(parts, dparts, h, Wr, Wu, Wc, br.reshape(1, U),
                bu.reshape(1, U), bc.reshape(1, U))


# speedup vs baseline: 1.2571x; 1.0892x over previous
import jax
import jax.numpy as jnp
from jax import lax
from jax.experimental import pallas as pl
from jax.experimental.pallas import tpu as pltpu
from jax.experimental.pallas import tpu_sc as plsc

N = 10000
E = 320000
D = 128
U = 64

NC = 2
NS = 16
L = 16
NW = NC * NS
EPW = E // NW
CH = 80
NCH = EPW // CH
WROW = U


def _prep_body(x_ref, w_ref, asrc_ref, adst_ref, mext_ref, asd_ref):
    m = jnp.dot(x_ref[...], w_ref[...], preferred_element_type=jnp.float32)
    a_s = jnp.sum(m * asrc_ref[...], axis=1)
    a_d = jnp.sum(m * adst_ref[...], axis=1)
    asd_ref[0, :] = a_s
    asd_ref[1, :] = a_d
    cb = jnp.max(a_s) + jnp.max(a_d)
    asd_ref[2, :] = jnp.full((N,), jnp.where(cb > 0.0, cb, 0.2 * cb),
                             jnp.float32)
    mext_ref[...] = m


def _prep(x, w, asrc, adst):
    return pl.pallas_call(
        _prep_body,
        out_shape=[
            jax.ShapeDtypeStruct((N, WROW), jnp.float32),
            jax.ShapeDtypeStruct((3, N), jnp.float32),
        ],
        out_specs=[
            pl.BlockSpec(memory_space=pltpu.VMEM),
            pl.BlockSpec(memory_space=pltpu.VMEM),
        ],
    )(x, w, asrc, adst)


def _edge_body(mext_hbm, asd_hbm, sd_hbm,
               out_hbm, outd_hbm,
               a_s_v, a_d_v, src_v, dst_v, ee_v, rows_v, rows2_v, rows3_v,
               denom_v, obuf_v, c_v, agg_sh,
               gsem, gsem2, gsem3, ssem, ssem2, ssem3):
    cid = lax.axis_index("c")
    sid = lax.axis_index("s")
    wid = cid * NS + sid

    pltpu.async_copy(asd_hbm.at[0], a_s_v, gsem)
    pltpu.async_copy(asd_hbm.at[1], a_d_v, gsem2)
    pltpu.async_copy(sd_hbm.at[0].at[wid], src_v, gsem3)
    pltpu.async_copy(sd_hbm.at[1].at[wid], dst_v, ssem)
    pltpu.async_copy(asd_hbm.at[2].at[pl.ds(0, L)], c_v, ssem2)
    pltpu.make_async_copy(asd_hbm.at[0], a_s_v, gsem).wait()
    pltpu.make_async_copy(asd_hbm.at[1], a_d_v, gsem2).wait()
    pltpu.make_async_copy(sd_hbm.at[0].at[wid], src_v, gsem3).wait()
    pltpu.make_async_copy(sd_hbm.at[1].at[wid], dst_v, ssem).wait()
    pltpu.make_async_copy(asd_hbm.at[2].at[pl.ds(0, L)], c_v, ssem2).wait()
    cvec = c_v[...]

    @plsc.parallel_loop(0, NCH, unroll=4)
    def _(r):
        for p in range(WROW // L):
            obuf_v[r, pl.ds(p * L, L)] = jnp.zeros((L,), jnp.float32)

    @plsc.parallel_loop(0, N // L, unroll=4)
    def _(r):
        denom_v[pl.ds(r * L, L)] = jnp.zeros((L,), jnp.float32)

    rows_per_tile = N // NS
    for t in range(rows_per_tile // NCH):
        pltpu.sync_copy(
            obuf_v, agg_sh.at[pl.ds(sid * rows_per_tile + t * NCH, NCH)])
    plsc.subcore_barrier()

    def ee_chunk(r):
        for p in range(CH // L):
            sl = pl.ds(p * L, L)
            si = src_v[r, sl]
            di = dst_v[r, sl]
            sv = plsc.load_gather(a_s_v, [si])
            dv = plsc.load_gather(a_d_v, [di])
            e = sv + dv
            e = jnp.where(e >= 0.0, e, 0.2 * e)
            ee = jnp.exp(e - cvec)
            ee_v[r, sl] = ee
            plsc.addupdate_scatter(denom_v, [di], ee)

    bufs = (rows_v, rows2_v, rows3_v)
    gsems = (gsem, gsem2, gsem3)
    ssems = (ssem, ssem2, ssem3)

    def g_issue(j, b):
        pltpu.async_copy(mext_hbm.at[src_v.at[j]], bufs[b], gsems[b])

    def s_wait(j, b):
        pltpu.make_async_copy(bufs[b], agg_sh.at[dst_v.at[j]],
                              ssems[b]).wait()

    def step(j, b, first=False, last_d=False):
        if last_d:
            @pl.when(j + 2 < NCH)
            def _():
                ee_chunk(j + 2)
        else:
            ee_chunk(j + 2)
        pltpu.make_async_copy(mext_hbm.at[src_v.at[j]], bufs[b],
                              gsems[b]).wait()
        rbuf = bufs[b]

        @plsc.parallel_loop(0, CH, unroll=4)
        def _(k):
            eek = plsc.load_gather(ee_v, [jnp.full((L,), j, jnp.int32),
                                          jnp.full((L,), k, jnp.int32)])
            for p in range(WROW // L):
                sl = pl.ds(p * L, L)
                rbuf[k, sl] = rbuf[k, sl] * eek

        pltpu.async_copy(rbuf, agg_sh.at[dst_v.at[j]], ssems[b], add=True)
        if not first:
            s_wait(j - 1, (b + 2) % 3)
        if last_d:
            @pl.when(j + 2 < NCH)
            def _():
                g_issue(j + 2, (b + 2) % 3)
        else:
            g_issue(j + 2, (b + 2) % 3)

    ee_chunk(0)
    ee_chunk(1)
    g_issue(0, 0)
    g_issue(1, 1)
    step(0, 0, first=True)

    @pl.loop(0, (NCH - 2) // 3)
    def _(t):
        j = 1 + 3 * t
        step(j, 1)
        step(j + 1, 2)
        step(j + 2, 0, last_d=True)

    step(NCH - 1, (NCH - 1) % 3, last_d=True)
    s_wait(NCH - 1, (NCH - 1) % 3)

    plsc.subcore_barrier()

    pltpu.async_copy(denom_v, outd_hbm.at[wid], gsem)
    for t in range(rows_per_tile // NCH):
        base = sid * rows_per_tile + t * NCH
        pltpu.async_copy(agg_sh.at[pl.ds(base, NCH)],
                         out_hbm.at[pl.ds(base, NCH), pl.ds(cid * U, U)],
                         ssems[t % 3])
    pltpu.make_async_copy(denom_v, outd_hbm.at[wid], gsem).wait()
    for t in range(rows_per_tile // NCH):
        base = sid * rows_per_tile + t * NCH
        pltpu.make_async_copy(agg_sh.at[pl.ds(base, NCH)],
                              out_hbm.at[pl.ds(base, NCH),
                                         pl.ds(cid * U, U)],
                              ssems[t % 3]).wait()


_edge = pl.kernel(
    _edge_body,
    out_type=[jax.ShapeDtypeStruct((N, NC * WROW), jnp.float32),
              jax.ShapeDtypeStruct((NW, N), jnp.float32)],
    mesh=plsc.VectorSubcoreMesh(core_axis_name="c", subcore_axis_name="s",
                                num_cores=NC, num_subcores=NS),
    scratch_types=[
        pltpu.VMEM((N,), jnp.float32),
        pltpu.VMEM((N,), jnp.float32),
        pltpu.VMEM((NCH, CH), jnp.int32),
        pltpu.VMEM((NCH, CH), jnp.int32),
        pltpu.VMEM((NCH, CH), jnp.float32),
        pltpu.VMEM((CH, WROW), jnp.float32),
        pltpu.VMEM((CH, WROW), jnp.float32),
        pltpu.VMEM((CH, WROW), jnp.float32),
        pltpu.VMEM((N,), jnp.float32),
        pltpu.VMEM((NCH, WROW), jnp.float32),
        pltpu.VMEM((L,), jnp.float32),
        pltpu.VMEM_SHARED((N, WROW), jnp.float32),
        pltpu.SemaphoreType.DMA,
        pltpu.SemaphoreType.DMA,
        pltpu.SemaphoreType.DMA,
        pltpu.SemaphoreType.DMA,
        pltpu.SemaphoreType.DMA,
        pltpu.SemaphoreType.DMA,
    ],
    compiler_params=pltpu.CompilerParams(use_tc_tiling_on_sc=False,
                                         needs_layout_passes=False),
)


def _gru_body(parts_ref, dpt_ref, h_ref, wr_ref, wu_ref, wc_ref, br_ref,
              bu_ref, bc_ref, out_ref):
    s = parts_ref[:, 0:U] + parts_ref[:, U:2 * U]
    den = jnp.sum(dpt_ref[...], axis=0)
    denom = den.reshape(s.shape[0], 1) + 1e-16
    agg = s / denom
    h = h_ref[...]
    xr = jnp.concatenate([agg, h], axis=1)
    r = jax.nn.sigmoid(
        jnp.dot(xr, wr_ref[...], preferred_element_type=jnp.float32)
        + br_ref[...])
    u = jax.nn.sigmoid(
        jnp.dot(xr, wu_ref[...], preferred_element_type=jnp.float32)
        + bu_ref[...])
    xc = jnp.concatenate([agg, r * h], axis=1)
    c = jnp.tanh(
        jnp.dot(xc, wc_ref[...], preferred_element_type=jnp.float32)
        + bc_ref[...])
    out_ref[...] = u * h + (1.0 - u) * c


def _gru(parts, dpt, h, wr, wu, wc, br, bu, bc):
    return pl.pallas_call(
        _gru_body,
        out_shape=jax.ShapeDtypeStruct((N, U), jnp.float32),
    )(parts, dpt, h, wr, wu, wc, br, bu, bc)


def kernel(x, h, edge_index, W, att_src, att_dst, Wr, Wu, Wc, br, bu, bc):
    mext, asd = _prep(x, W, att_src.reshape(1, U), att_dst.reshape(1, U))
    sd = edge_index.reshape(2, NW, NCH, CH)
    parts, dparts = _edge(mext, asd, sd)
    return _gru(parts, dparts, h, Wr, Wu, Wc, br.reshape(1, U),
                bu.reshape(1, U), bc.reshape(1, U))


# speedup vs baseline: 1.3128x; 1.0443x over previous
import jax
import jax.numpy as jnp
import numpy as np
from jax import lax
from jax.experimental import pallas as pl
from jax.experimental.pallas import tpu as pltpu
from jax.experimental.pallas import tpu_sc as plsc

N = 10000
E = 320000
D = 128
U = 64

NC = 2
NS = 16
L = 16
NW = NC * NS
EPW = E // NW
CH = 80
NCH = EPW // CH
WROW = U


def _prep_body(x_ref, w_ref, asrc_ref, adst_ref, mext_ref, asd_ref):
    m = jnp.dot(x_ref[...], w_ref[...], preferred_element_type=jnp.float32)
    a_s = jnp.sum(m * asrc_ref[...], axis=1)
    a_d = jnp.sum(m * adst_ref[...], axis=1)
    asd_ref[0, :] = a_s
    asd_ref[1, :] = a_d
    cb = jnp.max(a_s) + jnp.max(a_d)
    asd_ref[2, :] = jnp.full((N,), jnp.where(cb > 0.0, cb, 0.2 * cb),
                             jnp.float32)
    mext_ref[...] = m.astype(jnp.bfloat16)


def _prep(x, w, asrc, adst):
    return pl.pallas_call(
        _prep_body,
        out_shape=[
            jax.ShapeDtypeStruct((N, WROW), jnp.bfloat16),
            jax.ShapeDtypeStruct((3, N), jnp.float32),
        ],
        out_specs=[
            pl.BlockSpec(memory_space=pltpu.VMEM),
            pl.BlockSpec(memory_space=pltpu.VMEM),
        ],
    )(x, w, asrc, adst)


def _edge_body(mext_hbm, asd_hbm, sd_hbm,
               out_hbm, outd_hbm,
               a_s_v, a_d_v, src_v, dst_v, ee_v, rows_v, rows2_v, rows3_v,
               sc1_v, sc2_v, sc3_v, denom_v, obuf_v, c_v, agg_sh,
               gsem, gsem2, gsem3, ssem, ssem2, ssem3):
    cid = lax.axis_index("c")
    sid = lax.axis_index("s")
    wid = cid * NS + sid

    pltpu.async_copy(asd_hbm.at[0], a_s_v, gsem)
    pltpu.async_copy(asd_hbm.at[1], a_d_v, gsem2)
    pltpu.async_copy(sd_hbm.at[0].at[wid], src_v, gsem3)
    pltpu.async_copy(sd_hbm.at[1].at[wid], dst_v, ssem)
    pltpu.async_copy(asd_hbm.at[2].at[pl.ds(0, L)], c_v, ssem2)
    pltpu.make_async_copy(asd_hbm.at[0], a_s_v, gsem).wait()
    pltpu.make_async_copy(asd_hbm.at[1], a_d_v, gsem2).wait()
    pltpu.make_async_copy(sd_hbm.at[0].at[wid], src_v, gsem3).wait()
    pltpu.make_async_copy(sd_hbm.at[1].at[wid], dst_v, ssem).wait()
    pltpu.make_async_copy(asd_hbm.at[2].at[pl.ds(0, L)], c_v, ssem2).wait()
    cvec = c_v[...]

    @plsc.parallel_loop(0, NCH, unroll=4)
    def _(r):
        for p in range(WROW // L):
            obuf_v[r, pl.ds(p * L, L)] = jnp.zeros((L,), jnp.float32)

    @plsc.parallel_loop(0, N // L, unroll=4)
    def _(r):
        denom_v[pl.ds(r * L, L)] = jnp.zeros((L,), jnp.float32)

    rows_per_tile = N // NS
    for t in range(rows_per_tile // NCH):
        pltpu.sync_copy(
            obuf_v, agg_sh.at[pl.ds(sid * rows_per_tile + t * NCH, NCH)])
    plsc.subcore_barrier()

    def ee_chunk(r):
        for p in range(CH // L):
            sl = pl.ds(p * L, L)
            si = src_v[r, sl]
            di = dst_v[r, sl]
            sv = plsc.load_gather(a_s_v, [si])
            dv = plsc.load_gather(a_d_v, [di])
            e = sv + dv
            e = jnp.where(e >= 0.0, e, 0.2 * e)
            ee = jnp.exp(e - cvec)
            ee_v[r, sl] = ee
            plsc.addupdate_scatter(denom_v, [di], ee)

    bufs = (rows_v, rows2_v, rows3_v)
    sbufs = (sc1_v, sc2_v, sc3_v)
    gsems = (gsem, gsem2, gsem3)
    ssems = (ssem, ssem2, ssem3)

    def g_issue(j, b):
        pltpu.async_copy(mext_hbm.at[src_v.at[j]], bufs[b], gsems[b])

    def s_wait(j, b):
        pltpu.make_async_copy(sbufs[b], agg_sh.at[dst_v.at[j]],
                              ssems[b]).wait()

    def step(j, b, first=False, last_d=False):
        if last_d:
            @pl.when(j + 2 < NCH)
            def _():
                ee_chunk(j + 2)
        else:
            ee_chunk(j + 2)
        pltpu.make_async_copy(mext_hbm.at[src_v.at[j]], bufs[b],
                              gsems[b]).wait()
        rbuf = bufs[b]

        sbuf = sbufs[b]

        @plsc.parallel_loop(0, CH, unroll=4)
        def _(k):
            eek = plsc.load_gather(ee_v, [jnp.full((L,), j, jnp.int32),
                                          jnp.full((L,), k, jnp.int32)])
            for p in range(WROW // (2 * L)):
                v = rbuf[k, pl.ds(p * 2 * L, 2 * L)]
                a, bb = plsc.unpack(v, format=plsc.PackFormat.INTERLEAVED)
                sbuf[k, pl.ds(p * 2 * L, L)] = a * eek
                sbuf[k, pl.ds(p * 2 * L + L, L)] = bb * eek

        pltpu.async_copy(sbufs[b], agg_sh.at[dst_v.at[j]], ssems[b],
                         add=True)
        if not first:
            s_wait(j - 1, (b + 2) % 3)
        if last_d:
            @pl.when(j + 2 < NCH)
            def _():
                g_issue(j + 2, (b + 2) % 3)
        else:
            g_issue(j + 2, (b + 2) % 3)

    ee_chunk(0)
    ee_chunk(1)
    g_issue(0, 0)
    g_issue(1, 1)
    step(0, 0, first=True)

    @pl.loop(0, (NCH - 2) // 3)
    def _(t):
        j = 1 + 3 * t
        step(j, 1)
        step(j + 1, 2)
        step(j + 2, 0, last_d=True)

    step(NCH - 1, (NCH - 1) % 3, last_d=True)
    s_wait(NCH - 1, (NCH - 1) % 3)

    plsc.subcore_barrier()

    pltpu.async_copy(denom_v, outd_hbm.at[wid], gsem)
    for t in range(rows_per_tile // NCH):
        base = sid * rows_per_tile + t * NCH
        pltpu.async_copy(agg_sh.at[pl.ds(base, NCH)],
                         out_hbm.at[pl.ds(base, NCH), pl.ds(cid * U, U)],
                         ssems[t % 3])
    pltpu.make_async_copy(denom_v, outd_hbm.at[wid], gsem).wait()
    for t in range(rows_per_tile // NCH):
        base = sid * rows_per_tile + t * NCH
        pltpu.make_async_copy(agg_sh.at[pl.ds(base, NCH)],
                              out_hbm.at[pl.ds(base, NCH),
                                         pl.ds(cid * U, U)],
                              ssems[t % 3]).wait()


_edge = pl.kernel(
    _edge_body,
    out_type=[jax.ShapeDtypeStruct((N, NC * WROW), jnp.float32),
              jax.ShapeDtypeStruct((NW, N), jnp.float32)],
    mesh=plsc.VectorSubcoreMesh(core_axis_name="c", subcore_axis_name="s",
                                num_cores=NC, num_subcores=NS),
    scratch_types=[
        pltpu.VMEM((N,), jnp.float32),
        pltpu.VMEM((N,), jnp.float32),
        pltpu.VMEM((NCH, CH), jnp.int32),
        pltpu.VMEM((NCH, CH), jnp.int32),
        pltpu.VMEM((NCH, CH), jnp.float32),
        pltpu.VMEM((CH, WROW), jnp.bfloat16),
        pltpu.VMEM((CH, WROW), jnp.bfloat16),
        pltpu.VMEM((CH, WROW), jnp.bfloat16),
        pltpu.VMEM((CH, WROW), jnp.float32),
        pltpu.VMEM((CH, WROW), jnp.float32),
        pltpu.VMEM((CH, WROW), jnp.float32),
        pltpu.VMEM((N,), jnp.float32),
        pltpu.VMEM((NCH, WROW), jnp.float32),
        pltpu.VMEM((L,), jnp.float32),
        pltpu.VMEM_SHARED((N, WROW), jnp.float32),
        pltpu.SemaphoreType.DMA,
        pltpu.SemaphoreType.DMA,
        pltpu.SemaphoreType.DMA,
        pltpu.SemaphoreType.DMA,
        pltpu.SemaphoreType.DMA,
        pltpu.SemaphoreType.DMA,
    ],
    compiler_params=pltpu.CompilerParams(use_tc_tiling_on_sc=False,
                                         needs_layout_passes=False),
)


def _gru_body(parts_ref, dpt_ref, h_ref, wr_ref, wu_ref, wc_ref, br_ref,
              bu_ref, bc_ref, out_ref):
    s = parts_ref[:, 0:U] + parts_ref[:, U:2 * U]
    den = jnp.sum(dpt_ref[...], axis=0)
    denom = den.reshape(s.shape[0], 1) + 1e-16
    agg = s / denom
    h = h_ref[...]
    xr = jnp.concatenate([agg, h], axis=1)
    r = jax.nn.sigmoid(
        jnp.dot(xr, wr_ref[...], preferred_element_type=jnp.float32)
        + br_ref[...])
    u = jax.nn.sigmoid(
        jnp.dot(xr, wu_ref[...], preferred_element_type=jnp.float32)
        + bu_ref[...])
    xc = jnp.concatenate([agg, r * h], axis=1)
    c = jnp.tanh(
        jnp.dot(xc, wc_ref[...], preferred_element_type=jnp.float32)
        + bc_ref[...])
    out_ref[...] = u * h + (1.0 - u) * c


def _gru(parts, dpt, h, wr, wu, wc, br, bu, bc):
    return pl.pallas_call(
        _gru_body,
        out_shape=jax.ShapeDtypeStruct((N, U), jnp.float32),
    )(parts, dpt, h, wr, wu, wc, br, bu, bc)


_PERM = np.array([g * 2 * L + 2 * i + half
                  for g in range(U // (2 * L))
                  for half in range(2)
                  for i in range(L)][:U], dtype=np.int32)


def _permute_gate(wm):
    return jnp.concatenate([wm[:U][_PERM], wm[U:]], axis=0)


def kernel(x, h, edge_index, W, att_src, att_dst, Wr, Wu, Wc, br, bu, bc):
    Wr = _permute_gate(Wr)
    Wu = _permute_gate(Wu)
    Wc = _permute_gate(Wc)
    mext, asd = _prep(x, W, att_src.reshape(1, U), att_dst.reshape(1, U))
    sd = edge_index.reshape(2, NW, NCH, CH)
    parts, dparts = _edge(mext, asd, sd)
    return _gru(parts, dparts, h, Wr, Wu, Wc, br.reshape(1, U),
                bu.reshape(1, U), bc.reshape(1, U))
